# Initial kernel scaffold; baseline (speedup 1.0000x reference)
import jax
import jax.numpy as jnp
from jax.experimental import pallas as pl


def kernel(edge_index, emb_weight):
    raise NotImplementedError("write your pallas kernel here")


# speedup vs baseline: 5.3744x; 5.3744x over previous
import functools

import jax
import jax.numpy as jnp
from jax import lax
from jax.experimental import pallas as pl
from jax.experimental.pallas import tpu as pltpu
from jax.experimental.pallas import tpu_sc as plsc

_NUM_USERS = 20000
_NUM_ITEMS = 30000
_N = _NUM_USERS + _NUM_ITEMS
_D = 64
_L = 3
_E = 800000

_NC = 2
_NS = 16
_NPAD = 51200
_ROWS_PER_TILE = _NPAD // _NS
_CHUNK = 800
_SUB = 128
_NSUB = 4
_EB = _SUB * _NSUB
_ITERS = 100
_EP = _NS * _ITERS * _EB


def _sc_propagate(x0, src_g, dst_g):
    mesh = plsc.VectorSubcoreMesh(core_axis_name="c", subcore_axis_name="s")
    table = jax.ShapeDtypeStruct((_NC * _NPAD, 32), jnp.float32)

    @functools.partial(
        pl.kernel,
        out_type=(table, table, table),
        mesh=mesh,
        scratch_types=[
            pltpu.VMEM((_NSUB, _SUB), jnp.int32),
            pltpu.VMEM((_NSUB, _SUB), jnp.int32),
            pltpu.VMEM((_NSUB, _SUB, 32), jnp.float32),
            pltpu.VMEM_SHARED((_NPAD, 32), jnp.float32),
            pltpu.SemaphoreType.DMA,
        ],
        compiler_params=pltpu.CompilerParams(use_tc_tiling_on_sc=False),
    )
    def run(x0_hbm, src_hbm, dst_hbm, o1, o2, o3,
            src_v, dst_v, rows_v, acc, gsem):
        c = lax.axis_index("c")
        t = lax.axis_index("s")
        reg0 = t * _ROWS_PER_TILE

        for k in range(_ROWS_PER_TILE // _CHUNK):
            pltpu.sync_copy(x0_hbm.at[pl.ds(_N, _CHUNK)],
                            acc.at[pl.ds(reg0 + k * _CHUNK, _CHUNK)])
        plsc.subcore_barrier()

        def make_body(xin):
            def body(i, carry):
                pltpu.sync_copy(src_hbm.at[(c * _NS + t) * _ITERS + i], src_v)
                pltpu.sync_copy(dst_hbm.at[t * _ITERS + i], dst_v)
                descs = [
                    pltpu.async_copy(xin.at[src_v.at[j]], rows_v.at[j], gsem)
                    for j in range(_NSUB)
                ]
                for d in descs:
                    d.wait()
                for j in range(_NSUB):
                    pltpu.sync_copy(rows_v.at[j], acc.at[dst_v.at[j]], add=True)
                return carry
            return body

        outs = (o1, o2, o3)
        for l in range(_L):
            xin = x0_hbm if l == 0 else outs[l - 1]
            lax.fori_loop(0, _ITERS, make_body(xin), 0)
            plsc.subcore_barrier()
            for k in range(_ROWS_PER_TILE // _CHUNK):
                lo = reg0 + k * _CHUNK
                pltpu.sync_copy(acc.at[pl.ds(lo, _CHUNK)],
                                outs[l].at[pl.ds(c * _NPAD + lo, _CHUNK)])
                if l < _L - 1:
                    pltpu.sync_copy(x0_hbm.at[pl.ds(_N, _CHUNK)],
                                    acc.at[pl.ds(lo, _CHUNK)])
            plsc.subcore_barrier()

    return run(x0, src_g, dst_g)


def _combine_body(e_ref, a_ref, b_ref, c_ref, o_ref):
    left = (e_ref[:, :32] + a_ref[0] + b_ref[0] + c_ref[0]) * 0.25
    right = (e_ref[:, 32:] + a_ref[1] + b_ref[1] + c_ref[1]) * 0.25
    o_ref[:, :] = jnp.concatenate([left, right], axis=-1)


def _combine(emb, x1, x2, x3):
    blk = 400
    half_spec = pl.BlockSpec((2, blk, 32), lambda i: (0, i, 0))
    return pl.pallas_call(
        _combine_body,
        grid=(_N // blk,),
        in_specs=[pl.BlockSpec((blk, _D), lambda i: (i, 0)),
                  half_spec, half_spec, half_spec],
        out_specs=pl.BlockSpec((blk, _D), lambda i: (i, 0)),
        out_shape=jax.ShapeDtypeStruct((_N, _D), jnp.float32),
    )(emb, x1.reshape(_NC, _NPAD, 32), x2.reshape(_NC, _NPAD, 32),
      x3.reshape(_NC, _NPAD, 32))


def kernel(edge_index, emb_weight):
    src = edge_index[0]
    dst = edge_index[1]

    pad = _EP - _E
    src_p = jnp.concatenate([src, jnp.full((pad,), _N, jnp.int32)])
    dst_p = jnp.concatenate([dst, jnp.full((pad,), _N, jnp.int32)])
    src_g = jnp.stack([src_p, src_p + _NPAD]).reshape(
        _NC * _NS * _ITERS, _NSUB, _SUB)
    dst_g = dst_p.reshape(_NS * _ITERS, _NSUB, _SUB)

    x0 = jnp.zeros((_NC * _NPAD, 32), jnp.float32)
    x0 = x0.at[:_N, :].set(emb_weight[:, :32])
    x0 = x0.at[_NPAD:_NPAD + _N, :].set(emb_weight[:, 32:])

    x1, x2, x3 = _sc_propagate(x0, src_g, dst_g)
    final = _combine(emb_weight, x1, x2, x3)
    return (final[:_NUM_USERS], final[_NUM_USERS:])


# speedup vs baseline: 8.3277x; 1.5495x over previous
import functools

import jax
import jax.numpy as jnp
from jax import lax
from jax.experimental import pallas as pl
from jax.experimental.pallas import tpu as pltpu
from jax.experimental.pallas import tpu_sc as plsc

_NUM_USERS = 20000
_NUM_ITEMS = 30000
_N = _NUM_USERS + _NUM_ITEMS
_D = 64
_L = 3
_E = 800000

_NC = 2
_NS = 16
_NPAD = 50048
_ROWS_PER_TILE = _NPAD // _NS
_SUB = 128
_NSUB = 3
_EB = _SUB * _NSUB
_ITERS = 132
_EP = _NS * _ITERS * _EB


def _sc_propagate(x0, src_g, dst_g, zeros):
    mesh = plsc.VectorSubcoreMesh(core_axis_name="c", subcore_axis_name="s")
    table = jax.ShapeDtypeStruct((_NC * _NPAD, 32), jnp.float32)

    @functools.partial(
        pl.kernel,
        out_type=(table, table, table),
        mesh=mesh,
        scratch_types=[
            pltpu.VMEM((2, _NSUB, _SUB), jnp.int32),
            pltpu.VMEM((2, _NSUB, _SUB), jnp.int32),
            pltpu.VMEM((2, _NSUB, _SUB, 32), jnp.float32),
            pltpu.VMEM_SHARED((_NPAD, 32), jnp.float32),
            pltpu.SemaphoreType.DMA,
            pltpu.SemaphoreType.DMA,
            pltpu.SemaphoreType.DMA,
            pltpu.SemaphoreType.DMA,
        ],
        compiler_params=pltpu.CompilerParams(use_tc_tiling_on_sc=False),
    )
    def run(x0_hbm, src_hbm, dst_hbm, z_hbm, o1, o2, o3,
            src_v, dst_v, rows_v, acc, gsem, ssem, isem, osem):
        c = lax.axis_index("c")
        t = lax.axis_index("s")
        reg0 = t * _ROWS_PER_TILE

        pltpu.sync_copy(z_hbm, acc.at[pl.ds(reg0, _ROWS_PER_TILE)])
        plsc.subcore_barrier()

        def make_body(xin):
            def body(i2, carry):
                for p in range(2):
                    q, b = 1 - p, 2 * i2 + p
                    gd = [pltpu.async_copy(xin.at[src_v.at[p, j]],
                                           rows_v.at[p, j], gsem)
                          for j in range(_NSUB)]

                    @pl.when(b > 0)
                    def _():
                        for j in range(_NSUB):
                            pltpu.make_async_copy(
                                rows_v.at[q, j], acc.at[dst_v.at[q, j]],
                                ssem).wait()

                    @pl.when(b + 1 < _ITERS)
                    def _():
                        pltpu.async_copy(
                            src_hbm.at[(c * _NS + t) * _ITERS + b + 1],
                            src_v.at[q], isem)
                        pltpu.async_copy(
                            dst_hbm.at[t * _ITERS + b + 1], dst_v.at[q], isem)

                    for d in gd:
                        d.wait()

                    for j in range(_NSUB):
                        pltpu.async_copy(rows_v.at[p, j],
                                         acc.at[dst_v.at[p, j]], ssem,
                                         add=True)

                    @pl.when(b + 1 < _ITERS)
                    def _():
                        pltpu.make_async_copy(
                            src_hbm.at[(c * _NS + t) * _ITERS + b + 1],
                            src_v.at[q], isem).wait()
                        pltpu.make_async_copy(
                            dst_hbm.at[t * _ITERS + b + 1], dst_v.at[q],
                            isem).wait()
                return carry
            return body

        outs = (o1, o2, o3)
        for l in range(_L):
            xin = x0_hbm if l == 0 else outs[l - 1]
            pltpu.sync_copy(src_hbm.at[(c * _NS + t) * _ITERS], src_v.at[0])
            pltpu.sync_copy(dst_hbm.at[t * _ITERS], dst_v.at[0])
            lax.fori_loop(0, _ITERS // 2, make_body(xin), 0)
            for j in range(_NSUB):
                pltpu.make_async_copy(rows_v.at[1, j],
                                      acc.at[dst_v.at[1, j]], ssem).wait()
            plsc.subcore_barrier()
            pltpu.async_copy(acc.at[pl.ds(reg0, _ROWS_PER_TILE)],
                             outs[l].at[pl.ds(c * _NPAD + reg0,
                                              _ROWS_PER_TILE)], osem).wait()
            if l < _L - 1:
                pltpu.sync_copy(z_hbm, acc.at[pl.ds(reg0, _ROWS_PER_TILE)])
            plsc.subcore_barrier()

    return run(x0, src_g, dst_g, zeros)


def _combine_body(e_ref, a_ref, b_ref, c_ref, o_ref):
    left = (e_ref[:, :32] + a_ref[0] + b_ref[0] + c_ref[0]) * 0.25
    right = (e_ref[:, 32:] + a_ref[1] + b_ref[1] + c_ref[1]) * 0.25
    o_ref[:, :] = jnp.concatenate([left, right], axis=-1)


def _combine(emb, x1, x2, x3):
    blk = 400
    half_spec = pl.BlockSpec((2, blk, 32), lambda i: (0, i, 0))
    return pl.pallas_call(
        _combine_body,
        grid=(_N // blk,),
        in_specs=[pl.BlockSpec((blk, _D), lambda i: (i, 0)),
                  half_spec, half_spec, half_spec],
        out_specs=pl.BlockSpec((blk, _D), lambda i: (i, 0)),
        out_shape=jax.ShapeDtypeStruct((_N, _D), jnp.float32),
    )(emb, x1.reshape(_NC, _NPAD, 32), x2.reshape(_NC, _NPAD, 32),
      x3.reshape(_NC, _NPAD, 32))


def kernel(edge_index, emb_weight):
    src = edge_index[0]
    dst = edge_index[1]

    pad = _EP - _E
    src_p = jnp.concatenate([src, jnp.full((pad,), _N, jnp.int32)])
    dst_p = jnp.concatenate([dst, jnp.full((pad,), _N, jnp.int32)])
    src_g = jnp.stack([src_p, src_p + _NPAD]).reshape(
        _NC * _NS * _ITERS, _NSUB, _SUB)
    dst_g = dst_p.reshape(_NS * _ITERS, _NSUB, _SUB)

    x0 = jnp.zeros((_NC * _NPAD, 32), jnp.float32)
    x0 = x0.at[:_N, :].set(emb_weight[:, :32])
    x0 = x0.at[_NPAD:_NPAD + _N, :].set(emb_weight[:, 32:])
    zeros = jnp.zeros((_ROWS_PER_TILE, 32), jnp.float32)

    x1, x2, x3 = _sc_propagate(x0, src_g, dst_g, zeros)
    final = _combine(emb_weight, x1, x2, x3)
    return (final[:_NUM_USERS], final[_NUM_USERS:])


# speedup vs baseline: 8.3393x; 1.0014x over previous
import functools

import jax
import jax.numpy as jnp
from jax import lax
from jax.experimental import pallas as pl
from jax.experimental.pallas import tpu as pltpu
from jax.experimental.pallas import tpu_sc as plsc

_NUM_USERS = 20000
_NUM_ITEMS = 30000
_N = _NUM_USERS + _NUM_ITEMS
_D = 64
_L = 3
_E = 800000

_NC = 2
_NS = 16
_NPAD = 50048
_ROWS_PER_TILE = _NPAD // _NS
_SUB = 128
_NSUB = 3
_EB = _SUB * _NSUB
_ITERS = 132
_EP = _NS * _ITERS * _EB


def _sc_propagate(x0, src_g, dst_g, zeros):
    mesh = plsc.VectorSubcoreMesh(core_axis_name="c", subcore_axis_name="s")
    table = jax.ShapeDtypeStruct((_NC * _NPAD, 32), jnp.float32)

    @functools.partial(
        pl.kernel,
        out_type=(table, table, table),
        mesh=mesh,
        scratch_types=[
            pltpu.VMEM((2, _EB), jnp.int32),
            pltpu.VMEM((2, _NSUB, _SUB), jnp.int32),
            pltpu.VMEM((2, _EB, 32), jnp.float32),
            pltpu.VMEM_SHARED((_NPAD, 32), jnp.float32),
            pltpu.SemaphoreType.DMA,
            pltpu.SemaphoreType.DMA,
            pltpu.SemaphoreType.DMA,
            pltpu.SemaphoreType.DMA,
        ],
        compiler_params=pltpu.CompilerParams(use_tc_tiling_on_sc=False),
    )
    def run(x0_hbm, src_hbm, dst_hbm, z_hbm, o1, o2, o3,
            src_v, dst_v, rows_v, acc, gsem, ssem, isem, osem):
        c = lax.axis_index("c")
        t = lax.axis_index("s")
        reg0 = t * _ROWS_PER_TILE

        pltpu.sync_copy(z_hbm, acc.at[pl.ds(reg0, _ROWS_PER_TILE)])
        plsc.subcore_barrier()

        def make_body(xin):
            def body(i2, carry):
                for p in range(2):
                    q, b = 1 - p, 2 * i2 + p
                    gd = pltpu.async_copy(xin.at[src_v.at[p]],
                                          rows_v.at[p], gsem)

                    @pl.when(b > 0)
                    def _():
                        for j in range(_NSUB):
                            pltpu.make_async_copy(
                                rows_v.at[q, pl.ds(j * _SUB, _SUB)],
                                acc.at[dst_v.at[q, j]], ssem).wait()

                    @pl.when(b + 1 < _ITERS)
                    def _():
                        pltpu.async_copy(
                            src_hbm.at[(c * _NS + t) * _ITERS + b + 1],
                            src_v.at[q], isem)
                        pltpu.async_copy(
                            dst_hbm.at[t * _ITERS + b + 1], dst_v.at[q], isem)

                    gd.wait()

                    for j in range(_NSUB):
                        pltpu.async_copy(rows_v.at[p, pl.ds(j * _SUB, _SUB)],
                                         acc.at[dst_v.at[p, j]], ssem,
                                         add=True)

                    @pl.when(b + 1 < _ITERS)
                    def _():
                        pltpu.make_async_copy(
                            src_hbm.at[(c * _NS + t) * _ITERS + b + 1],
                            src_v.at[q], isem).wait()
                        pltpu.make_async_copy(
                            dst_hbm.at[t * _ITERS + b + 1], dst_v.at[q],
                            isem).wait()
                return carry
            return body

        outs = (o1, o2, o3)
        for l in range(_L):
            xin = x0_hbm if l == 0 else outs[l - 1]
            pltpu.sync_copy(src_hbm.at[(c * _NS + t) * _ITERS], src_v.at[0])
            pltpu.sync_copy(dst_hbm.at[t * _ITERS], dst_v.at[0])
            lax.fori_loop(0, _ITERS // 2, make_body(xin), 0)
            for j in range(_NSUB):
                pltpu.make_async_copy(rows_v.at[1, pl.ds(j * _SUB, _SUB)],
                                      acc.at[dst_v.at[1, j]], ssem).wait()
            plsc.subcore_barrier()
            pltpu.async_copy(acc.at[pl.ds(reg0, _ROWS_PER_TILE)],
                             outs[l].at[pl.ds(c * _NPAD + reg0,
                                              _ROWS_PER_TILE)], osem).wait()
            if l < _L - 1:
                pltpu.sync_copy(z_hbm, acc.at[pl.ds(reg0, _ROWS_PER_TILE)])
            plsc.subcore_barrier()

    return run(x0, src_g, dst_g, zeros)


def _combine_body(e_ref, a_ref, b_ref, c_ref, o_ref):
    left = (e_ref[:, :32] + a_ref[0] + b_ref[0] + c_ref[0]) * 0.25
    right = (e_ref[:, 32:] + a_ref[1] + b_ref[1] + c_ref[1]) * 0.25
    o_ref[:, :] = jnp.concatenate([left, right], axis=-1)


def _combine(emb, x1, x2, x3):
    blk = 400
    half_spec = pl.BlockSpec((2, blk, 32), lambda i: (0, i, 0))
    return pl.pallas_call(
        _combine_body,
        grid=(_N // blk,),
        in_specs=[pl.BlockSpec((blk, _D), lambda i: (i, 0)),
                  half_spec, half_spec, half_spec],
        out_specs=pl.BlockSpec((blk, _D), lambda i: (i, 0)),
        out_shape=jax.ShapeDtypeStruct((_N, _D), jnp.float32),
    )(emb, x1.reshape(_NC, _NPAD, 32), x2.reshape(_NC, _NPAD, 32),
      x3.reshape(_NC, _NPAD, 32))


def kernel(edge_index, emb_weight):
    src = edge_index[0]
    dst = edge_index[1]

    pad = _EP - _E
    src_p = jnp.concatenate([src, jnp.full((pad,), _N, jnp.int32)])
    dst_p = jnp.concatenate([dst, jnp.full((pad,), _N, jnp.int32)])
    src_g = jnp.stack([src_p, src_p + _NPAD]).reshape(
        _NC * _NS * _ITERS, _EB)
    dst_g = dst_p.reshape(_NS * _ITERS, _NSUB, _SUB)

    x0 = jnp.zeros((_NC * _NPAD, 32), jnp.float32)
    x0 = x0.at[:_N, :].set(emb_weight[:, :32])
    x0 = x0.at[_NPAD:_NPAD + _N, :].set(emb_weight[:, 32:])
    zeros = jnp.zeros((_ROWS_PER_TILE, 32), jnp.float32)

    x1, x2, x3 = _sc_propagate(x0, src_g, dst_g, zeros)
    final = _combine(emb_weight, x1, x2, x3)
    return (final[:_NUM_USERS], final[_NUM_USERS:])
